# Initial kernel scaffold; baseline (speedup 1.0000x reference)
import jax
import jax.numpy as jnp
from jax.experimental import pallas as pl


def kernel(sparse_features, dense_features, W0, W_sparse, W_dense_w, W_dense_b, V_sparse, V_dense_w, V_dense_b):
    raise NotImplementedError("write your pallas kernel here")


# speedup vs baseline: 2.4146x; 2.4146x over previous
import functools

import jax
import jax.numpy as jnp
from jax import lax
from jax.experimental import pallas as pl
from jax.experimental.pallas import tpu as pltpu
from jax.experimental.pallas import tpu_sc as plsc

_NC = 2
_NS = 16
_NW = _NC * _NS
_LANES = 16

_B = 16384
_F = 26
_K = 32

_RPT = _B // _NW
_CH = 32
_NCH = _RPT // _CH
_IDXM = 104
_IPC = _CH * _F // _IDXM
_IDX_ROWS = _RPT * _F // _IDXM
_CHI = _CH * _F


def _sc_body(idx_hbm, v_hbm, w_hbm, s_out, sq_out, wraw_out,
             idxv, vb0, vb1, wall, sbuf, sqbuf, sem0, sem1, wsem):
    wid = lax.axis_index("s") * _NC + lax.axis_index("c")

    pltpu.sync_copy(idx_hbm.at[pl.ds(wid * _IDX_ROWS, _IDX_ROWS)], idxv)

    def fire(c, vb, sem):
        for j in range(_IPC):
            row = c * _IPC + j
            pltpu.async_copy(v_hbm.at[idxv.at[row]],
                             vb.at[pl.ds(j * _IDXM, _IDXM)], sem)
            pltpu.async_copy(w_hbm.at[idxv.at[row]],
                             wall.at[pl.ds(c * _CHI + j * _IDXM, _IDXM)],
                             wsem)

    def drain(vb, sem):
        pltpu.make_async_copy(v_hbm.at[pl.ds(0, _CHI)], vb, sem).wait()

    def compute(c, vb):
        def row_body(r, carry):
            rb = r * _F
            acc0 = jnp.zeros((_LANES,), jnp.float32)
            acc1 = jnp.zeros((_LANES,), jnp.float32)
            asq = jnp.zeros((_LANES,), jnp.float32)
            for f in range(_F):
                v0 = vb[rb + f, 0:16]
                v1 = vb[rb + f, 16:32]
                acc0 = acc0 + v0
                acc1 = acc1 + v1
                asq = asq + v0 * v0
                asq = asq + v1 * v1
            gr = c * _CH + r
            sbuf[gr, 0:16] = acc0
            sbuf[gr, 16:32] = acc1
            sqbuf[gr, :] = asq
            return carry

        lax.fori_loop(0, _CH, row_body, 0)

    bufs = ((vb0, sem0), (vb1, sem1))
    fire(0, vb0, sem0)

    def chunk_body(i, carry):
        for b in range(2):
            c = i * 2 + b
            vb, sem = bufs[b]
            nvb, nsem = bufs[1 - b]

            @pl.when(c + 1 < _NCH)
            def _():
                fire(c + 1, nvb, nsem)

            drain(vb, sem)
            compute(c, vb)
        return carry

    lax.fori_loop(0, _NCH // 2, chunk_body, 0)

    base = wid * _RPT
    pltpu.sync_copy(sbuf, s_out.at[pl.ds(base, _RPT)])
    pltpu.sync_copy(sqbuf, sq_out.at[pl.ds(base, _RPT)])
    pltpu.make_async_copy(w_hbm.at[pl.ds(0, _RPT * _F)], wall, wsem).wait()
    pltpu.sync_copy(wall, wraw_out.at[pl.ds(wid * _RPT * _F, _RPT * _F)])


_sc_gather = functools.partial(
    pl.kernel,
    mesh=plsc.VectorSubcoreMesh(core_axis_name="c", subcore_axis_name="s"),
    compiler_params=pltpu.CompilerParams(use_tc_tiling_on_sc=False),
    out_type=[
        jax.ShapeDtypeStruct((_B, _K), jnp.float32),
        jax.ShapeDtypeStruct((_B, _LANES), jnp.float32),
        jax.ShapeDtypeStruct((_B * _F,), jnp.float32),
    ],
    scratch_types=[
        pltpu.VMEM((_IDX_ROWS, _IDXM), jnp.int32),
        pltpu.VMEM((_CHI, _K), jnp.float32),
        pltpu.VMEM((_CHI, _K), jnp.float32),
        pltpu.VMEM((_RPT * _F,), jnp.float32),
        pltpu.VMEM((_RPT, _K), jnp.float32),
        pltpu.VMEM((_RPT, _LANES), jnp.float32),
        pltpu.SemaphoreType.DMA,
        pltpu.SemaphoreType.DMA,
        pltpu.SemaphoreType.DMA,
    ],
)(_sc_body)


def _tc_body(s_ref, sq_ref, wraw_ref, dense_ref, w0_ref, wdw_ref, wdb_ref,
             vdw_ref, vdb_ref, out_ref):
    dense = dense_ref[:]
    d = lax.dot_general(dense, vdw_ref[:], (((1,), (1,)), ((), ())),
                        preferred_element_type=jnp.float32) + vdb_ref[:]
    t = s_ref[:] + d
    second = (jnp.sum(t * t, axis=1, keepdims=True)
              - jnp.sum(sq_ref[:], axis=1, keepdims=True)
              - jnp.sum(d * d, axis=1, keepdims=True))
    first_sparse = jnp.sum(wraw_ref[:], axis=1, keepdims=True)
    first_dense = lax.dot_general(dense, wdw_ref[:], (((1,), (1,)), ((), ())),
                                  preferred_element_type=jnp.float32)
    out_ref[:] = (w0_ref[:] + first_sparse + first_dense + wdb_ref[:]
                  + 0.5 * second)


def kernel(sparse_features, dense_features, W0, W_sparse, W_dense_w,
           W_dense_b, V_sparse, V_dense_w, V_dense_b):
    idx = sparse_features.astype(jnp.int32).reshape(_B * _F // _IDXM, _IDXM)
    w_flat = W_sparse.reshape(-1)

    s, sq16, wraw = _sc_gather(idx, V_sparse, w_flat)

    blk = 2048
    grid = (_B // blk,)
    out = pl.pallas_call(
        _tc_body,
        grid=grid,
        in_specs=[
            pl.BlockSpec((blk, _K), lambda i: (i, 0)),
            pl.BlockSpec((blk, _LANES), lambda i: (i, 0)),
            pl.BlockSpec((blk, _F), lambda i: (i, 0)),
            pl.BlockSpec((blk, dense_features.shape[1]), lambda i: (i, 0)),
            pl.BlockSpec((1, 1), lambda i: (0, 0)),
            pl.BlockSpec(W_dense_w.shape, lambda i: (0, 0)),
            pl.BlockSpec((1, 1), lambda i: (0, 0)),
            pl.BlockSpec(V_dense_w.shape, lambda i: (0, 0)),
            pl.BlockSpec((1, _K), lambda i: (0, 0)),
        ],
        out_specs=pl.BlockSpec((blk, 1), lambda i: (i, 0)),
        out_shape=jax.ShapeDtypeStruct((_B, 1), jnp.float32),
    )(s, sq16, wraw.reshape(_B, _F), dense_features, W0.reshape(1, 1),
      W_dense_w, W_dense_b.reshape(1, 1), V_dense_w,
      V_dense_b.reshape(1, _K))
    return out
